# Initial kernel scaffold; baseline (speedup 1.0000x reference)
import jax
import jax.numpy as jnp
from jax.experimental import pallas as pl


def kernel(features1, edge_index1, features2, edge_index2, params):
    raise NotImplementedError("write your pallas kernel here")


# speedup vs baseline: 2.8725x; 2.8725x over previous
import functools

import jax
import jax.numpy as jnp
from jax import lax
from jax.experimental import pallas as pl
from jax.experimental.pallas import tpu as pltpu
from jax.experimental.pallas import tpu_sc as plsc

N = 10000
E = 320000
D = 128

NC = 2
NS = 16
NW = NC * NS
C = 128
CHUNKS = -(-E // (NW * C))
EPW = CHUNKS * C
E_PAD = NW * EPW
ZR = 32
R = NS * (-(-(N + 1) // (NS * ZR))) * ZR
ROWS_PT = R // NS

_MESH = plsc.VectorSubcoreMesh(core_axis_name="c", subcore_axis_name="s")


def _zero_vmem(ref, rows, width):
    def body(r, _):
        for c in range(width // 16):
            ref[r, pl.ds(c * 16, 16)] = jnp.zeros((16,), jnp.float32)
        return 0
    lax.fori_loop(0, rows, body, 0)


def _agg_body(with_count, x_hbm, src_hbm, dst_hbm, *rest):
    if with_count:
        (out_hbm, cnt_hbm, src_v, dst_v, rows_v, bounce_v, accum,
         ones_v, cbounce_v, caccum, sem) = rest
    else:
        (out_hbm, src_v, dst_v, rows_v, bounce_v, accum, sem) = rest

    cid = lax.axis_index("c")
    sid = lax.axis_index("s")
    wid = sid * NC + cid

    _zero_vmem(bounce_v, ZR, D)
    zbase = sid * ROWS_PT
    for j in range(ROWS_PT // ZR):
        pltpu.sync_copy(bounce_v, accum.at[pl.ds(zbase + j * ZR, ZR)])
    if with_count:
        for k in range(C // 16):
            ones_v[pl.ds(k * 16, 16)] = jnp.ones((16,), jnp.float32)
        for k in range(ROWS_PT // 16):
            cbounce_v[pl.ds(k * 16, 16)] = jnp.zeros((16,), jnp.float32)
        pltpu.sync_copy(cbounce_v, caccum.at[pl.ds(zbase, ROWS_PT)])
    plsc.subcore_barrier()

    def chunk_body(i, _):
        base = pl.multiple_of(wid * EPW + i * C, 8)
        pltpu.sync_copy(src_hbm.at[pl.ds(base, C)], src_v)
        pltpu.sync_copy(dst_hbm.at[pl.ds(base, C)], dst_v)
        pltpu.async_copy(x_hbm.at[src_v], rows_v, sem).wait()
        pltpu.sync_copy(rows_v, accum.at[dst_v], add=True)
        if with_count:
            pltpu.sync_copy(ones_v, caccum.at[dst_v], add=True)
        return 0
    lax.fori_loop(0, CHUNKS, chunk_body, 0)
    plsc.subcore_barrier()

    for j in range(ROWS_PT // ZR):
        off = zbase + j * ZR
        pltpu.sync_copy(accum.at[pl.ds(off, ZR)], bounce_v)
        pltpu.sync_copy(bounce_v, out_hbm.at[cid, pl.ds(off, ZR)])
    if with_count:
        pltpu.sync_copy(caccum.at[pl.ds(zbase, ROWS_PT)], cbounce_v)
        pltpu.sync_copy(cbounce_v, cnt_hbm.at[cid, pl.ds(zbase, ROWS_PT)])


def _make_agg(with_count):
    out_type = [jax.ShapeDtypeStruct((NC, R, D), jnp.float32)]
    scratch = [
        pltpu.VMEM((C,), jnp.int32),
        pltpu.VMEM((C,), jnp.int32),
        pltpu.VMEM((C, D), jnp.float32),
        pltpu.VMEM((ZR, D), jnp.float32),
        pltpu.VMEM_SHARED((R, D), jnp.float32),
    ]
    if with_count:
        out_type.append(jax.ShapeDtypeStruct((NC, R), jnp.float32))
        scratch += [
            pltpu.VMEM((C,), jnp.float32),
            pltpu.VMEM((ROWS_PT,), jnp.float32),
            pltpu.VMEM_SHARED((R,), jnp.float32),
        ]
    scratch.append(pltpu.SemaphoreType.DMA)
    return pl.kernel(
        functools.partial(_agg_body, with_count),
        out_type=tuple(out_type) if with_count else out_type[0],
        mesh=_MESH,
        scratch_types=scratch,
    )


_agg_sum = _make_agg(False)
_agg_sum_count = _make_agg(True)


BLK = 1000
_CONTRACT = (((1,), (1,)), ((), ()))


def _sage_combine(s, ic, x, wl, bl, wr):
    mean = s * ic
    acc = lax.dot_general(mean, wl, _CONTRACT,
                          preferred_element_type=jnp.float32)
    acc = acc + lax.dot_general(x, wr, _CONTRACT,
                                preferred_element_type=jnp.float32)
    acc = acc + bl
    nrm = jnp.sqrt(jnp.sum(acc * acc, axis=1, keepdims=True))
    return acc / jnp.maximum(nrm, 1e-12)


def _layer_first_body(s_ref, cnt_ref, x_ref, wl_ref, bl_ref, wr_ref,
                      o_ref, ic_ref):
    cnt = cnt_ref[0, :, 0:1] + cnt_ref[1, :, 0:1]
    ic = 1.0 / jnp.maximum(cnt, 1.0)
    ic_ref[...] = ic
    s = s_ref[0] + s_ref[1]
    o_ref[...] = _sage_combine(s, ic, x_ref[...], wl_ref[...], bl_ref[...],
                               wr_ref[...])


def _layer_body(s_ref, ic_ref, x_ref, wl_ref, bl_ref, wr_ref, o_ref):
    s = s_ref[0] + s_ref[1]
    o_ref[...] = _sage_combine(s, ic_ref[...], x_ref[...], wl_ref[...],
                               bl_ref[...], wr_ref[...])


def _fc_body(x1_ref, x2_ref, w_ref, b_ref, o_ref):
    w = w_ref[...]
    acc = lax.dot_general(x1_ref[...], w[:, :D], _CONTRACT,
                          preferred_element_type=jnp.float32)
    acc = acc + lax.dot_general(x2_ref[...], w[:, D:], _CONTRACT,
                                preferred_element_type=jnp.float32)
    o_ref[...] = acc + b_ref[...]


_W_SPEC = pl.BlockSpec((D, D), lambda i: (0, 0))
_B_SPEC = pl.BlockSpec((1, D), lambda i: (0, 0))
_X_SPEC = pl.BlockSpec((BLK, D), lambda i: (i, 0))
_S_SPEC = pl.BlockSpec((NC, BLK, D), lambda i: (0, i, 0))
_IC_SPEC = pl.BlockSpec((BLK, 1), lambda i: (i, 0))

_layer_first = pl.pallas_call(
    _layer_first_body,
    grid=(N // BLK,),
    in_specs=[_S_SPEC, pl.BlockSpec((NC, BLK, 1), lambda i: (0, i, 0)),
              _X_SPEC, _W_SPEC, _B_SPEC, _W_SPEC],
    out_specs=[_X_SPEC, _IC_SPEC],
    out_shape=[jax.ShapeDtypeStruct((N, D), jnp.float32),
               jax.ShapeDtypeStruct((N, 1), jnp.float32)],
)

_layer = pl.pallas_call(
    _layer_body,
    grid=(N // BLK,),
    in_specs=[_S_SPEC, _IC_SPEC, _X_SPEC, _W_SPEC, _B_SPEC, _W_SPEC],
    out_specs=_X_SPEC,
    out_shape=jax.ShapeDtypeStruct((N, D), jnp.float32),
)

_fc = pl.pallas_call(
    _fc_body,
    grid=(N // BLK,),
    in_specs=[_X_SPEC, _X_SPEC, pl.BlockSpec((D, 2 * D), lambda i: (0, 0)),
              _B_SPEC],
    out_specs=_X_SPEC,
    out_shape=jax.ShapeDtypeStruct((N, D), jnp.float32),
)


def _pad_edges(ei):
    pad = E_PAD - E
    src = jnp.concatenate([ei[0], jnp.zeros((pad,), jnp.int32)])
    dst = jnp.concatenate([ei[1], jnp.full((pad,), N, jnp.int32)])
    return src, dst


def kernel(features1, edge_index1, features2, edge_index2, params):
    p = params
    src1, dst1 = _pad_edges(edge_index1)
    src2, dst2 = _pad_edges(edge_index2)

    def wargs(nm):
        return (p[nm + '_Wl'], p[nm + '_bl'].reshape(1, D), p[nm + '_Wr'])

    s, c1 = _agg_sum_count(features1, src1, dst1)
    h, ic1 = _layer_first(s, c1.reshape(NC, R, 1), features1, *wargs('e1c1'))
    s = _agg_sum(h, src1, dst1)
    x1 = _layer(s, ic1, h, *wargs('e1c2'))
    s, c2 = _agg_sum_count(features2, src2, dst2)
    h, ic2 = _layer_first(s, c2.reshape(NC, R, 1), features2, *wargs('e2c1'))
    s = _agg_sum(h, src2, dst2)
    x2 = _layer(s, ic2, h, *wargs('e2c2'))
    x = _fc(x1, x2, p['fc_W'], p['fc_b'].reshape(1, D))
    s = _agg_sum(x, src1, dst1)
    h = _layer(s, ic1, x, *wargs('d1c1'))
    s = _agg_sum(h, src1, dst1)
    x1_rec = _layer(s, ic1, h, *wargs('d1c2'))
    s = _agg_sum(x, src2, dst2)
    h = _layer(s, ic2, x, *wargs('d2c1'))
    s = _agg_sum(h, src2, dst2)
    x2_rec = _layer(s, ic2, h, *wargs('d2c2'))
    return (x, x1_rec, x2_rec)
